# Initial kernel scaffold; baseline (speedup 1.0000x reference)
import jax
import jax.numpy as jnp
from jax.experimental import pallas as pl


def kernel(x, edge_index, edge_weight, W_tag1, b_tag1, W_tag2, b_tag2, W_deca, b_deca, W_bil, W_dx1, b_dx1, W_dx2, b_dx2, W_dx3, b_dx3, W_pi, b_pi, W_disp, b_disp, W_mean, b_mean):
    raise NotImplementedError("write your pallas kernel here")


# speedup vs baseline: 5.8532x; 5.8532x over previous
import functools

import jax
import jax.numpy as jnp
from jax import lax
from jax.experimental import pallas as pl
from jax.experimental.pallas import tpu as pltpu
from jax.experimental.pallas import tpu_sc as plsc

_N = 10000
_E = 160000
_NW = 32
_RPT = 624


def _make_prop(D, chunk):
    nch_total = _E // chunk
    nch_base = nch_total // _NW
    nch_extra = nch_total % _NW
    mesh = plsc.VectorSubcoreMesh(core_axis_name="c", subcore_axis_name="s",
                                  num_cores=2, num_subcores=16)

    @functools.partial(
        pl.kernel,
        out_type=jax.ShapeDtypeStruct((2, _N, D), jnp.float32),
        mesh=mesh,
        compiler_params=pltpu.CompilerParams(use_tc_tiling_on_sc=(D == 128)),
        scratch_types=[
            pltpu.VMEM((chunk,), jnp.int32),
            pltpu.VMEM((chunk,), jnp.int32),
            pltpu.VMEM((chunk,), jnp.float32),
            pltpu.VMEM((chunk, D), jnp.float32),
            pltpu.VMEM_SHARED((_N, D), jnp.float32),
            pltpu.SemaphoreType.DMA,
        ],
    )
    def prop(table, src, dst, w, zeros, out, src_v, dst_v, w_v, rows_v,
             acc_sh, sem):
        cid = lax.axis_index("c")
        sid = lax.axis_index("s")
        wid = cid * 16 + sid
        r0 = sid * _RPT
        pltpu.sync_copy(zeros.at[pl.ds(r0, _RPT)], acc_sh.at[pl.ds(r0, _RPT)])

        @pl.when(sid == 15)
        def _():
            tail = 16 * _RPT
            pltpu.sync_copy(zeros.at[pl.ds(tail, _N - tail)],
                            acc_sh.at[pl.ds(tail, _N - tail)])

        plsc.subcore_barrier()

        def step(i, carry):
            base = (wid + _NW * i) * chunk
            pltpu.sync_copy(src.at[pl.ds(base, chunk)], src_v)
            pltpu.sync_copy(dst.at[pl.ds(base, chunk)], dst_v)
            pltpu.sync_copy(w.at[pl.ds(base, chunk)], w_v)
            pltpu.async_copy(table.at[src_v], rows_v, sem).wait()

            def scale16(j, carry2):
                w16 = w_v[pl.ds(j * 16, 16)]
                for e in range(16):
                    ws = jnp.full((16,), w16[e], dtype=jnp.float32)
                    row = j * 16 + e
                    for c in range(D // 16):
                        rows_v[row, pl.ds(c * 16, 16)] = (
                            rows_v[row, pl.ds(c * 16, 16)] * ws)
                return carry2

            lax.fori_loop(0, chunk // 16, scale16, 0)
            pltpu.sync_copy(rows_v, acc_sh.at[dst_v], add=True)
            return carry

        ntrips = nch_base + jnp.where(wid < nch_extra, 1, 0)
        lax.fori_loop(0, ntrips, step, 0)
        plsc.subcore_barrier()
        pltpu.sync_copy(acc_sh.at[pl.ds(r0, _RPT)],
                        out.at[cid, pl.ds(r0, _RPT)])

        @pl.when(sid == 15)
        def _():
            tail = 16 * _RPT
            pltpu.sync_copy(acc_sh.at[pl.ds(tail, _N - tail)],
                            out.at[cid, pl.ds(tail, _N - tail)])

    return prop


@functools.lru_cache(maxsize=None)
def _get_prop(D, chunk):
    return _make_prop(D, chunk)


def _prop128(*args):
    return _get_prop(128, 320)(*args)


def _prop16(*args):
    return _get_prop(16, 640)(*args)


def _merge2(p):
    R = 1000

    def body(p_ref, o_ref):
        o_ref[...] = p_ref[0] + p_ref[1]

    return pl.pallas_call(
        body,
        grid=(_N // R,),
        in_specs=[pl.BlockSpec((2, R, 128), lambda i: (0, i, 0))],
        out_specs=pl.BlockSpec((R, 128), lambda i: (i, 0)),
        out_shape=jax.ShapeDtypeStruct((_N, 128), jnp.float32),
    )(p)


def _tag1_final(x, h1, h2, p3, W1, b1, W2c):
    R = 1000

    def body(x_ref, h1_ref, h2_ref, p3_ref, W1_ref, b1_ref, W2c_ref,
             z0_ref, z1_ref, z2_ref, z3_ref):
        hop3 = p3_ref[0] + p3_ref[1]
        acc = x_ref[...] @ W1_ref[0:128]
        acc += h1_ref[...] @ W1_ref[128:256]
        acc += h2_ref[...] @ W1_ref[256:384]
        acc += hop3 @ W1_ref[384:512]
        h = jnp.maximum(acc + b1_ref[...], 0.0)
        zw = h @ W2c_ref[...]
        z0_ref[...] = zw[:, 0:16]
        z1_ref[...] = zw[:, 16:32]
        z2_ref[...] = zw[:, 32:48]
        z3_ref[...] = zw[:, 48:64]

    zspec = pl.BlockSpec((R, 16), lambda i: (i, 0))
    zshape = jax.ShapeDtypeStruct((_N, 16), jnp.float32)
    return pl.pallas_call(
        body,
        grid=(_N // R,),
        in_specs=[
            pl.BlockSpec((R, 128), lambda i: (i, 0)),
            pl.BlockSpec((R, 128), lambda i: (i, 0)),
            pl.BlockSpec((R, 128), lambda i: (i, 0)),
            pl.BlockSpec((2, R, 128), lambda i: (0, i, 0)),
            pl.BlockSpec((512, 128), lambda i: (0, 0)),
            pl.BlockSpec((1, 128), lambda i: (0, 0)),
            pl.BlockSpec((128, 64), lambda i: (0, 0)),
        ],
        out_specs=[zspec, zspec, zspec, zspec],
        out_shape=[zshape, zshape, zshape, zshape],
    )(x, h1, h2, p3, W1, b1, W2c)


def _add3_16(a, p):
    R = 2000

    def body(a_ref, p_ref, o_ref):
        o_ref[...] = a_ref[...] + p_ref[0] + p_ref[1]

    return pl.pallas_call(
        body,
        grid=(_N // R,),
        in_specs=[
            pl.BlockSpec((R, 16), lambda i: (i, 0)),
            pl.BlockSpec((2, R, 16), lambda i: (0, i, 0)),
        ],
        out_specs=pl.BlockSpec((R, 16), lambda i: (i, 0)),
        out_shape=jax.ShapeDtypeStruct((_N, 16), jnp.float32),
    )(a, p)


def _final_enc(zw0, q1, b2, Wd, bd, Wb):
    R = 2000

    def body(zw0_ref, q1_ref, b2_ref, Wd_ref, bd_ref, Wb_ref,
             z_ref, ha_ref, haB_ref):
        zf = zw0_ref[...] + q1_ref[0] + q1_ref[1]
        z15 = zf[:, 0:15] + b2_ref[...]
        z_ref[...] = z15
        ha = z15 @ Wd_ref[...] + bd_ref[...]
        ha_ref[...] = ha
        haB_ref[...] = ha @ Wb_ref[...]

    return pl.pallas_call(
        body,
        grid=(_N // R,),
        in_specs=[
            pl.BlockSpec((R, 16), lambda i: (i, 0)),
            pl.BlockSpec((2, R, 16), lambda i: (0, i, 0)),
            pl.BlockSpec((1, 15), lambda i: (0, 0)),
            pl.BlockSpec((15, 32), lambda i: (0, 0)),
            pl.BlockSpec((1, 32), lambda i: (0, 0)),
            pl.BlockSpec((32, 32), lambda i: (0, 0)),
        ],
        out_specs=[
            pl.BlockSpec((R, 15), lambda i: (i, 0)),
            pl.BlockSpec((R, 32), lambda i: (i, 0)),
            pl.BlockSpec((R, 32), lambda i: (i, 0)),
        ],
        out_shape=[
            jax.ShapeDtypeStruct((_N, 15), jnp.float32),
            jax.ShapeDtypeStruct((_N, 32), jnp.float32),
            jax.ShapeDtypeStruct((_N, 32), jnp.float32),
        ],
    )(zw0, q1, b2, Wd, bd, Wb)


def _decA(haB, haT):
    R, C = 1000, 1024

    def body(haB_ref, haT_ref, o_ref):
        o_ref[...] = jax.nn.sigmoid(
            jnp.dot(haB_ref[...], haT_ref[...],
                    preferred_element_type=jnp.float32))

    return pl.pallas_call(
        body,
        grid=(_N // R, pl.cdiv(_N, C)),
        in_specs=[
            pl.BlockSpec((R, 32), lambda i, j: (i, 0)),
            pl.BlockSpec((32, C), lambda i, j: (0, j)),
        ],
        out_specs=pl.BlockSpec((R, C), lambda i, j: (i, j)),
        out_shape=jax.ShapeDtypeStruct((_N, _N), jnp.float32),
    )(haB, haT)


def _decX(z, W1, b1, W2, b2, W3, b3, Wpi, bpi, Wdp, bdp, Wmu, bmu):
    R = 1000

    def body(z_ref, W1_ref, b1_ref, W2_ref, b2_ref, W3_ref, b3_ref,
             Wpi_ref, bpi_ref, Wdp_ref, bdp_ref, Wmu_ref, bmu_ref,
             pi_ref, dp_ref, mu_ref):
        h = jnp.maximum(z_ref[...] @ W1_ref[...] + b1_ref[...], 0.0)
        h = jnp.maximum(h @ W2_ref[...] + b2_ref[...], 0.0)
        h = jnp.maximum(h @ W3_ref[...] + b3_ref[...], 0.0)
        pi_ref[...] = jax.nn.sigmoid(h @ Wpi_ref[...] + bpi_ref[...])
        sp = jax.nn.softplus(h @ Wdp_ref[...] + bdp_ref[...])
        dp_ref[...] = jnp.clip(sp, 1e-4, 1e4)
        mu_ref[...] = jnp.clip(jnp.exp(h @ Wmu_ref[...] + bmu_ref[...]),
                               1e-5, 1e6)

    full = lambda a, b: pl.BlockSpec((a, b), lambda i: (0, 0))
    ospec = pl.BlockSpec((R, 128), lambda i: (i, 0))
    oshape = jax.ShapeDtypeStruct((_N, 128), jnp.float32)
    return pl.pallas_call(
        body,
        grid=(_N // R,),
        in_specs=[
            pl.BlockSpec((R, 15), lambda i: (i, 0)),
            full(15, 128), full(1, 128),
            full(128, 256), full(1, 256),
            full(256, 512), full(1, 512),
            full(512, 128), full(1, 128),
            full(512, 128), full(1, 128),
            full(512, 128), full(1, 128),
        ],
        out_specs=[ospec, ospec, ospec],
        out_shape=[oshape, oshape, oshape],
    )(z, W1, b1, W2, b2, W3, b3, Wpi, bpi, Wdp, bdp, Wmu, bmu)


def kernel(x, edge_index, edge_weight, W_tag1, b_tag1, W_tag2, b_tag2,
           W_deca, b_deca, W_bil, W_dx1, b_dx1, W_dx2, b_dx2, W_dx3, b_dx3,
           W_pi, b_pi, W_disp, b_disp, W_mean, b_mean):
    src = edge_index[0].astype(jnp.int32)
    dst = edge_index[1].astype(jnp.int32)
    w = edge_weight
    z128 = jnp.zeros((_N, 128), jnp.float32)
    z16 = jnp.zeros((_N, 16), jnp.float32)

    p1 = _prop128(x, src, dst, w, z128)
    hop1 = _merge2(p1)
    p2 = _prop128(hop1, src, dst, w, z128)
    hop2 = _merge2(p2)
    p3 = _prop128(hop2, src, dst, w, z128)

    W2c = jnp.pad(
        W_tag2.reshape(4, 128, 15).transpose(1, 0, 2),
        ((0, 0), (0, 0), (0, 1))).reshape(128, 64)
    zw0, zw1, zw2, zw3 = _tag1_final(
        x, hop1, hop2, p3, W_tag1, b_tag1.reshape(1, 128), W2c)

    q3 = _prop16(zw3, src, dst, w, z16)
    m3 = _add3_16(zw2, q3)
    q2 = _prop16(m3, src, dst, w, z16)
    m2 = _add3_16(zw1, q2)
    q1 = _prop16(m2, src, dst, w, z16)
    z, ha, haB = _final_enc(zw0, q1, b_tag2.reshape(1, 15), W_deca,
                            b_deca.reshape(1, 32), W_bil)

    A_out = _decA(haB, ha.T)
    pi, disp, mean = _decX(
        z, W_dx1, b_dx1.reshape(1, 128), W_dx2, b_dx2.reshape(1, 256),
        W_dx3, b_dx3.reshape(1, 512), W_pi, b_pi.reshape(1, 128),
        W_disp, b_disp.reshape(1, 128), W_mean, b_mean.reshape(1, 128))
    return (z, A_out, pi, disp, mean)


# speedup vs baseline: 6.5761x; 1.1235x over previous
import functools

import jax
import jax.numpy as jnp
from jax import lax
from jax.experimental import pallas as pl
from jax.experimental.pallas import tpu as pltpu
from jax.experimental.pallas import tpu_sc as plsc

_N = 10000
_E = 160000
_NW = 32
_RPT = 624


def _make_prop(D, chunk):
    nch_total = _E // chunk
    nch_base = nch_total // _NW
    nch_extra = nch_total % _NW
    mesh = plsc.VectorSubcoreMesh(core_axis_name="c", subcore_axis_name="s",
                                  num_cores=2, num_subcores=16)

    @functools.partial(
        pl.kernel,
        out_type=jax.ShapeDtypeStruct((2, _N, D), jnp.float32),
        mesh=mesh,
        compiler_params=pltpu.CompilerParams(use_tc_tiling_on_sc=(D == 128)),
        scratch_types=[
            pltpu.VMEM((chunk,), jnp.int32),
            pltpu.VMEM((chunk,), jnp.int32),
            pltpu.VMEM((chunk,), jnp.int32),
            pltpu.VMEM((chunk,), jnp.int32),
            pltpu.VMEM((chunk,), jnp.float32),
            pltpu.VMEM((chunk,), jnp.float32),
            pltpu.VMEM((chunk, D), jnp.float32),
            pltpu.VMEM((chunk, D), jnp.float32),
            pltpu.VMEM_SHARED((_N, D), jnp.float32),
            pltpu.SemaphoreType.DMA,
            pltpu.SemaphoreType.DMA,
            pltpu.SemaphoreType.DMA,
            pltpu.SemaphoreType.DMA,
        ],
    )
    def prop(table, src, dst, w, zeros, out, src0, src1, dst0, dst1, w0, w1,
             rows0, rows1, acc_sh, sg0, sg1, ss0, ss1):
        src_s = (src0, src1)
        dst_s = (dst0, dst1)
        w_s = (w0, w1)
        rows_b = (rows0, rows1)
        sem_g = (sg0, sg1)
        sem_s = (ss0, ss1)
        cid = lax.axis_index("c")
        sid = lax.axis_index("s")
        wid = cid * 16 + sid
        ntrips = nch_base + jnp.where(wid < nch_extra, 1, 0)

        def load_idx(i, b):
            base = (wid + _NW * i) * chunk
            pltpu.sync_copy(src.at[pl.ds(base, chunk)], src_s[b])
            pltpu.sync_copy(dst.at[pl.ds(base, chunk)], dst_s[b])
            pltpu.sync_copy(w.at[pl.ds(base, chunk)], w_s[b])

        def gather_desc(b):
            return pltpu.make_async_copy(table.at[src_s[b]], rows_b[b],
                                         sem_g[b])

        def scatter_desc(b):
            return pltpu.make_async_copy(rows_b[b], acc_sh.at[dst_s[b]],
                                         sem_s[b])

        def issue_scatter(b):
            pltpu.async_copy(rows_b[b], acc_sh.at[dst_s[b]], sem_s[b],
                             add=True)

        def scale(b):
            rv = rows_b[b]
            wref = w_s[b]

            def scale16(j, carry):
                w16 = wref[pl.ds(pl.multiple_of(j * 16, 8), 16)]
                for e in range(16):
                    ws = jnp.full((16,), w16[e], dtype=jnp.float32)
                    row = j * 16 + e
                    for c in range(D // 16):
                        rv[row, pl.ds(c * 16, 16)] = (
                            rv[row, pl.ds(c * 16, 16)] * ws)
                return carry

            lax.fori_loop(0, chunk // 16, scale16, 0)

        r0 = sid * _RPT
        pltpu.sync_copy(zeros.at[pl.ds(r0, _RPT)], acc_sh.at[pl.ds(r0, _RPT)])

        @pl.when(sid == 15)
        def _():
            tail = 16 * _RPT
            pltpu.sync_copy(zeros.at[pl.ds(tail, _N - tail)],
                            acc_sh.at[pl.ds(tail, _N - tail)])

        plsc.subcore_barrier()

        load_idx(0, 0)
        gather_desc(0).start()

        def step(i, carry):
            for b in (0, 1):
                @pl.when(i % 2 == b)
                def _(b=b):
                    nb = 1 - b

                    @pl.when(jnp.logical_and(i >= 1, i + 1 < ntrips))
                    def _():
                        scatter_desc(nb).wait()

                    @pl.when(i + 1 < ntrips)
                    def _():
                        load_idx(i + 1, nb)
                        gather_desc(nb).start()

                    gather_desc(b).wait()
                    scale(b)
                    issue_scatter(b)
            return carry

        lax.fori_loop(0, ntrips, step, 0)
        scatter_desc(0).wait()
        scatter_desc(1).wait()

        plsc.subcore_barrier()
        pltpu.sync_copy(acc_sh.at[pl.ds(r0, _RPT)],
                        out.at[cid, pl.ds(r0, _RPT)])

        @pl.when(sid == 15)
        def _():
            tail = 16 * _RPT
            pltpu.sync_copy(acc_sh.at[pl.ds(tail, _N - tail)],
                            out.at[cid, pl.ds(tail, _N - tail)])

    return prop


@functools.lru_cache(maxsize=None)
def _get_prop(D, chunk):
    return _make_prop(D, chunk)


def _prop128(*args):
    return _get_prop(128, 160)(*args)


def _prop16(*args):
    return _get_prop(16, 1280)(*args)


def _merge2(p):
    R = 1000

    def body(p_ref, o_ref):
        o_ref[...] = p_ref[0] + p_ref[1]

    return pl.pallas_call(
        body,
        grid=(_N // R,),
        in_specs=[pl.BlockSpec((2, R, 128), lambda i: (0, i, 0))],
        out_specs=pl.BlockSpec((R, 128), lambda i: (i, 0)),
        out_shape=jax.ShapeDtypeStruct((_N, 128), jnp.float32),
    )(p)


def _tag1_final(x, h1, h2, p3, W1, b1, W2c):
    R = 1000

    def body(x_ref, h1_ref, h2_ref, p3_ref, W1_ref, b1_ref, W2c_ref,
             z0_ref, z1_ref, z2_ref, z3_ref):
        hop3 = p3_ref[0] + p3_ref[1]
        acc = x_ref[...] @ W1_ref[0:128]
        acc += h1_ref[...] @ W1_ref[128:256]
        acc += h2_ref[...] @ W1_ref[256:384]
        acc += hop3 @ W1_ref[384:512]
        h = jnp.maximum(acc + b1_ref[...], 0.0)
        zw = h @ W2c_ref[...]
        z0_ref[...] = zw[:, 0:16]
        z1_ref[...] = zw[:, 16:32]
        z2_ref[...] = zw[:, 32:48]
        z3_ref[...] = zw[:, 48:64]

    zspec = pl.BlockSpec((R, 16), lambda i: (i, 0))
    zshape = jax.ShapeDtypeStruct((_N, 16), jnp.float32)
    return pl.pallas_call(
        body,
        grid=(_N // R,),
        in_specs=[
            pl.BlockSpec((R, 128), lambda i: (i, 0)),
            pl.BlockSpec((R, 128), lambda i: (i, 0)),
            pl.BlockSpec((R, 128), lambda i: (i, 0)),
            pl.BlockSpec((2, R, 128), lambda i: (0, i, 0)),
            pl.BlockSpec((512, 128), lambda i: (0, 0)),
            pl.BlockSpec((1, 128), lambda i: (0, 0)),
            pl.BlockSpec((128, 64), lambda i: (0, 0)),
        ],
        out_specs=[zspec, zspec, zspec, zspec],
        out_shape=[zshape, zshape, zshape, zshape],
    )(x, h1, h2, p3, W1, b1, W2c)


def _add3_16(a, p):
    R = 2000

    def body(a_ref, p_ref, o_ref):
        o_ref[...] = a_ref[...] + p_ref[0] + p_ref[1]

    return pl.pallas_call(
        body,
        grid=(_N // R,),
        in_specs=[
            pl.BlockSpec((R, 16), lambda i: (i, 0)),
            pl.BlockSpec((2, R, 16), lambda i: (0, i, 0)),
        ],
        out_specs=pl.BlockSpec((R, 16), lambda i: (i, 0)),
        out_shape=jax.ShapeDtypeStruct((_N, 16), jnp.float32),
    )(a, p)


def _final_enc(zw0, q1, b2, Wd, bd, Wb):
    R = 2000

    def body(zw0_ref, q1_ref, b2_ref, Wd_ref, bd_ref, Wb_ref,
             z_ref, ha_ref, haB_ref):
        zf = zw0_ref[...] + q1_ref[0] + q1_ref[1]
        z15 = zf[:, 0:15] + b2_ref[...]
        z_ref[...] = z15
        ha = z15 @ Wd_ref[...] + bd_ref[...]
        ha_ref[...] = ha.astype(jnp.bfloat16)
        haB_ref[...] = (ha @ Wb_ref[...]).astype(jnp.bfloat16)

    return pl.pallas_call(
        body,
        grid=(_N // R,),
        in_specs=[
            pl.BlockSpec((R, 16), lambda i: (i, 0)),
            pl.BlockSpec((2, R, 16), lambda i: (0, i, 0)),
            pl.BlockSpec((1, 15), lambda i: (0, 0)),
            pl.BlockSpec((15, 32), lambda i: (0, 0)),
            pl.BlockSpec((1, 32), lambda i: (0, 0)),
            pl.BlockSpec((32, 32), lambda i: (0, 0)),
        ],
        out_specs=[
            pl.BlockSpec((R, 15), lambda i: (i, 0)),
            pl.BlockSpec((R, 32), lambda i: (i, 0)),
            pl.BlockSpec((R, 32), lambda i: (i, 0)),
        ],
        out_shape=[
            jax.ShapeDtypeStruct((_N, 15), jnp.float32),
            jax.ShapeDtypeStruct((_N, 32), jnp.bfloat16),
            jax.ShapeDtypeStruct((_N, 32), jnp.bfloat16),
        ],
    )(zw0, q1, b2, Wd, bd, Wb)


def _decA(haB, haT):
    R, C = 1024, 1024

    def body(haB_ref, haT_ref, o_ref):
        o_ref[...] = jax.nn.sigmoid(
            jnp.dot(haB_ref[...], haT_ref[...],
                    preferred_element_type=jnp.float32))

    return pl.pallas_call(
        body,
        grid=(pl.cdiv(_N, R), pl.cdiv(_N, C)),
        in_specs=[
            pl.BlockSpec((R, 32), lambda i, j: (i, 0)),
            pl.BlockSpec((32, C), lambda i, j: (0, j)),
        ],
        out_specs=pl.BlockSpec((R, C), lambda i, j: (i, j)),
        out_shape=jax.ShapeDtypeStruct((_N, _N), jnp.float32),
    )(haB, haT)


def _decX(z, W1, b1, W2, b2, W3, b3, Wpi, bpi, Wdp, bdp, Wmu, bmu):
    R = 1000

    def body(z_ref, W1_ref, b1_ref, W2_ref, b2_ref, W3_ref, b3_ref,
             Wpi_ref, bpi_ref, Wdp_ref, bdp_ref, Wmu_ref, bmu_ref,
             pi_ref, dp_ref, mu_ref):
        h = jnp.maximum(z_ref[...] @ W1_ref[...] + b1_ref[...], 0.0)
        h = jnp.maximum(h @ W2_ref[...] + b2_ref[...], 0.0)
        h = jnp.maximum(h @ W3_ref[...] + b3_ref[...], 0.0)
        pi_ref[...] = jax.nn.sigmoid(h @ Wpi_ref[...] + bpi_ref[...])
        sp = jax.nn.softplus(h @ Wdp_ref[...] + bdp_ref[...])
        dp_ref[...] = jnp.clip(sp, 1e-4, 1e4)
        mu_ref[...] = jnp.clip(jnp.exp(h @ Wmu_ref[...] + bmu_ref[...]),
                               1e-5, 1e6)

    full = lambda a, b: pl.BlockSpec((a, b), lambda i: (0, 0))
    ospec = pl.BlockSpec((R, 128), lambda i: (i, 0))
    oshape = jax.ShapeDtypeStruct((_N, 128), jnp.float32)
    return pl.pallas_call(
        body,
        grid=(_N // R,),
        in_specs=[
            pl.BlockSpec((R, 15), lambda i: (i, 0)),
            full(15, 128), full(1, 128),
            full(128, 256), full(1, 256),
            full(256, 512), full(1, 512),
            full(512, 128), full(1, 128),
            full(512, 128), full(1, 128),
            full(512, 128), full(1, 128),
        ],
        out_specs=[ospec, ospec, ospec],
        out_shape=[oshape, oshape, oshape],
    )(z, W1, b1, W2, b2, W3, b3, Wpi, bpi, Wdp, bdp, Wmu, bmu)


def kernel(x, edge_index, edge_weight, W_tag1, b_tag1, W_tag2, b_tag2,
           W_deca, b_deca, W_bil, W_dx1, b_dx1, W_dx2, b_dx2, W_dx3, b_dx3,
           W_pi, b_pi, W_disp, b_disp, W_mean, b_mean):
    src = edge_index[0].astype(jnp.int32)
    dst = edge_index[1].astype(jnp.int32)
    w = edge_weight
    z128 = jnp.zeros((_N, 128), jnp.float32)
    z16 = jnp.zeros((_N, 16), jnp.float32)

    p1 = _prop128(x, src, dst, w, z128)
    hop1 = _merge2(p1)
    p2 = _prop128(hop1, src, dst, w, z128)
    hop2 = _merge2(p2)
    p3 = _prop128(hop2, src, dst, w, z128)

    W2c = jnp.pad(
        W_tag2.reshape(4, 128, 15).transpose(1, 0, 2),
        ((0, 0), (0, 0), (0, 1))).reshape(128, 64)
    zw0, zw1, zw2, zw3 = _tag1_final(
        x, hop1, hop2, p3, W_tag1, b_tag1.reshape(1, 128), W2c)

    q3 = _prop16(zw3, src, dst, w, z16)
    m3 = _add3_16(zw2, q3)
    q2 = _prop16(m3, src, dst, w, z16)
    m2 = _add3_16(zw1, q2)
    q1 = _prop16(m2, src, dst, w, z16)
    z, ha, haB = _final_enc(zw0, q1, b_tag2.reshape(1, 15), W_deca,
                            b_deca.reshape(1, 32), W_bil)

    A_out = _decA(haB, ha.T)
    pi, disp, mean = _decX(
        z, W_dx1, b_dx1.reshape(1, 128), W_dx2, b_dx2.reshape(1, 256),
        W_dx3, b_dx3.reshape(1, 512), W_pi, b_pi.reshape(1, 128),
        W_disp, b_disp.reshape(1, 128), W_mean, b_mean.reshape(1, 128))
    return (z, A_out, pi, disp, mean)


# speedup vs baseline: 7.1352x; 1.0850x over previous
import functools

import jax
import jax.numpy as jnp
from jax import lax
from jax.experimental import pallas as pl
from jax.experimental.pallas import tpu as pltpu
from jax.experimental.pallas import tpu_sc as plsc

_N = 10000
_E = 160000
_NW = 32
_RPT = 624


def _make_prop(D, chunk):
    nch_total = _E // chunk
    nch_base = nch_total // _NW
    nch_extra = nch_total % _NW
    mesh = plsc.VectorSubcoreMesh(core_axis_name="c", subcore_axis_name="s",
                                  num_cores=2, num_subcores=16)

    @functools.partial(
        pl.kernel,
        out_type=jax.ShapeDtypeStruct((2, _N, D), jnp.float32),
        mesh=mesh,
        compiler_params=pltpu.CompilerParams(use_tc_tiling_on_sc=(D == 128)),
        scratch_types=[
            pltpu.VMEM((chunk,), jnp.int32),
            pltpu.VMEM((chunk,), jnp.int32),
            pltpu.VMEM((chunk,), jnp.int32),
            pltpu.VMEM((chunk,), jnp.int32),
            pltpu.VMEM((chunk,), jnp.float32),
            pltpu.VMEM((chunk,), jnp.float32),
            pltpu.VMEM((chunk, D), jnp.float32),
            pltpu.VMEM((chunk, D), jnp.float32),
            pltpu.VMEM_SHARED((_N, D), jnp.float32),
            pltpu.SemaphoreType.DMA,
            pltpu.SemaphoreType.DMA,
            pltpu.SemaphoreType.DMA,
            pltpu.SemaphoreType.DMA,
            pltpu.SemaphoreType.DMA,
            pltpu.SemaphoreType.DMA,
        ],
    )
    def prop(table, src, dst, w, zeros, out, src0, src1, dst0, dst1, w0, w1,
             rows0, rows1, acc_sh, sg0, sg1, ss0, ss1, si0, si1):
        src_s = (src0, src1)
        dst_s = (dst0, dst1)
        w_s = (w0, w1)
        rows_b = (rows0, rows1)
        sem_g = (sg0, sg1)
        sem_s = (ss0, ss1)
        sem_i = (si0, si1)
        cid = lax.axis_index("c")
        sid = lax.axis_index("s")
        wid = cid * 16 + sid
        ntrips = nch_base + jnp.where(wid < nch_extra, 1, 0)

        def load_idx(i, b):
            base = (wid + _NW * i) * chunk
            pltpu.sync_copy(src.at[pl.ds(base, chunk)], src_s[b])
            pltpu.sync_copy(dst.at[pl.ds(base, chunk)], dst_s[b])
            pltpu.sync_copy(w.at[pl.ds(base, chunk)], w_s[b])

        def idx_descs(i, b):
            base = (wid + _NW * i) * chunk
            return (
                pltpu.make_async_copy(src.at[pl.ds(base, chunk)], src_s[b],
                                      sem_i[b]),
                pltpu.make_async_copy(dst.at[pl.ds(base, chunk)], dst_s[b],
                                      sem_i[b]),
                pltpu.make_async_copy(w.at[pl.ds(base, chunk)], w_s[b],
                                      sem_i[b]),
            )

        def gather_desc(b):
            return pltpu.make_async_copy(table.at[src_s[b]], rows_b[b],
                                         sem_g[b])

        def scatter_desc(b):
            return pltpu.make_async_copy(rows_b[b], acc_sh.at[dst_s[b]],
                                         sem_s[b])

        def issue_scatter(b):
            pltpu.async_copy(rows_b[b], acc_sh.at[dst_s[b]], sem_s[b],
                             add=True)

        def scale(b):
            rv = rows_b[b]
            wref = w_s[b]

            def scale16(j, carry):
                w16 = wref[pl.ds(pl.multiple_of(j * 16, 8), 16)]
                for e in range(16):
                    ws = jnp.full((16,), w16[e], dtype=jnp.float32)
                    row = j * 16 + e
                    for c in range(D // 16):
                        rv[row, pl.ds(c * 16, 16)] = (
                            rv[row, pl.ds(c * 16, 16)] * ws)
                return carry

            lax.fori_loop(0, chunk // 16, scale16, 0)

        r0 = sid * _RPT
        pltpu.sync_copy(zeros.at[pl.ds(r0, _RPT)], acc_sh.at[pl.ds(r0, _RPT)])

        @pl.when(sid == 15)
        def _():
            tail = 16 * _RPT
            pltpu.sync_copy(zeros.at[pl.ds(tail, _N - tail)],
                            acc_sh.at[pl.ds(tail, _N - tail)])

        plsc.subcore_barrier()

        load_idx(0, 0)
        gather_desc(0).start()

        def step(i, carry):
            for b in (0, 1):
                @pl.when(i % 2 == b)
                def _(b=b):
                    nb = 1 - b

                    @pl.when(jnp.logical_and(i >= 1, i + 1 < ntrips))
                    def _():
                        scatter_desc(nb).wait()

                    @pl.when(i + 1 < ntrips)
                    def _():
                        for dsc in idx_descs(i + 1, nb):
                            dsc.start()

                    gather_desc(b).wait()
                    scale(b)
                    issue_scatter(b)

                    @pl.when(i + 1 < ntrips)
                    def _():
                        for dsc in idx_descs(i + 1, nb):
                            dsc.wait()
                        gather_desc(nb).start()
            return carry

        lax.fori_loop(0, ntrips, step, 0)
        scatter_desc(0).wait()
        scatter_desc(1).wait()

        plsc.subcore_barrier()
        pltpu.sync_copy(acc_sh.at[pl.ds(r0, _RPT)],
                        out.at[cid, pl.ds(r0, _RPT)])

        @pl.when(sid == 15)
        def _():
            tail = 16 * _RPT
            pltpu.sync_copy(acc_sh.at[pl.ds(tail, _N - tail)],
                            out.at[cid, pl.ds(tail, _N - tail)])

    return prop


@functools.lru_cache(maxsize=None)
def _get_prop(D, chunk):
    return _make_prop(D, chunk)


def _prop128(*args):
    return _get_prop(128, 160)(*args)


def _prop16(*args):
    return _get_prop(16, 1280)(*args)


def _merge2(p):
    R = 1000

    def body(p_ref, o_ref):
        o_ref[...] = p_ref[0] + p_ref[1]

    return pl.pallas_call(
        body,
        grid=(_N // R,),
        in_specs=[pl.BlockSpec((2, R, 128), lambda i: (0, i, 0))],
        out_specs=pl.BlockSpec((R, 128), lambda i: (i, 0)),
        out_shape=jax.ShapeDtypeStruct((_N, 128), jnp.float32),
    )(p)


def _tag1_final(x, h1, h2, p3, W1, b1, W2c):
    R = 1000

    def body(x_ref, h1_ref, h2_ref, p3_ref, W1_ref, b1_ref, W2c_ref,
             z0_ref, z1_ref, z2_ref, z3_ref):
        hop3 = p3_ref[0] + p3_ref[1]
        acc = x_ref[...] @ W1_ref[0:128]
        acc += h1_ref[...] @ W1_ref[128:256]
        acc += h2_ref[...] @ W1_ref[256:384]
        acc += hop3 @ W1_ref[384:512]
        h = jnp.maximum(acc + b1_ref[...], 0.0)
        zw = h @ W2c_ref[...]
        z0_ref[...] = zw[:, 0:16]
        z1_ref[...] = zw[:, 16:32]
        z2_ref[...] = zw[:, 32:48]
        z3_ref[...] = zw[:, 48:64]

    zspec = pl.BlockSpec((R, 16), lambda i: (i, 0))
    zshape = jax.ShapeDtypeStruct((_N, 16), jnp.float32)
    return pl.pallas_call(
        body,
        grid=(_N // R,),
        in_specs=[
            pl.BlockSpec((R, 128), lambda i: (i, 0)),
            pl.BlockSpec((R, 128), lambda i: (i, 0)),
            pl.BlockSpec((R, 128), lambda i: (i, 0)),
            pl.BlockSpec((2, R, 128), lambda i: (0, i, 0)),
            pl.BlockSpec((512, 128), lambda i: (0, 0)),
            pl.BlockSpec((1, 128), lambda i: (0, 0)),
            pl.BlockSpec((128, 64), lambda i: (0, 0)),
        ],
        out_specs=[zspec, zspec, zspec, zspec],
        out_shape=[zshape, zshape, zshape, zshape],
    )(x, h1, h2, p3, W1, b1, W2c)


def _add3_16(a, p):
    R = 2000

    def body(a_ref, p_ref, o_ref):
        o_ref[...] = a_ref[...] + p_ref[0] + p_ref[1]

    return pl.pallas_call(
        body,
        grid=(_N // R,),
        in_specs=[
            pl.BlockSpec((R, 16), lambda i: (i, 0)),
            pl.BlockSpec((2, R, 16), lambda i: (0, i, 0)),
        ],
        out_specs=pl.BlockSpec((R, 16), lambda i: (i, 0)),
        out_shape=jax.ShapeDtypeStruct((_N, 16), jnp.float32),
    )(a, p)


def _final_enc(zw0, q1, b2, Wd, bd, Wb):
    R = 2000

    def body(zw0_ref, q1_ref, b2_ref, Wd_ref, bd_ref, Wb_ref,
             z_ref, ha_ref, haB_ref):
        zf = zw0_ref[...] + q1_ref[0] + q1_ref[1]
        z15 = zf[:, 0:15] + b2_ref[...]
        z_ref[...] = z15
        ha = z15 @ Wd_ref[...] + bd_ref[...]
        ha_ref[...] = ha.astype(jnp.bfloat16)
        haB_ref[...] = (ha @ Wb_ref[...]).astype(jnp.bfloat16)

    return pl.pallas_call(
        body,
        grid=(_N // R,),
        in_specs=[
            pl.BlockSpec((R, 16), lambda i: (i, 0)),
            pl.BlockSpec((2, R, 16), lambda i: (0, i, 0)),
            pl.BlockSpec((1, 15), lambda i: (0, 0)),
            pl.BlockSpec((15, 32), lambda i: (0, 0)),
            pl.BlockSpec((1, 32), lambda i: (0, 0)),
            pl.BlockSpec((32, 32), lambda i: (0, 0)),
        ],
        out_specs=[
            pl.BlockSpec((R, 15), lambda i: (i, 0)),
            pl.BlockSpec((R, 32), lambda i: (i, 0)),
            pl.BlockSpec((R, 32), lambda i: (i, 0)),
        ],
        out_shape=[
            jax.ShapeDtypeStruct((_N, 15), jnp.float32),
            jax.ShapeDtypeStruct((_N, 32), jnp.bfloat16),
            jax.ShapeDtypeStruct((_N, 32), jnp.bfloat16),
        ],
    )(zw0, q1, b2, Wd, bd, Wb)


def _decA(haB, haT):
    R, C = 1024, 1024

    def body(haB_ref, haT_ref, o_ref):
        o_ref[...] = jax.nn.sigmoid(
            jnp.dot(haB_ref[...], haT_ref[...],
                    preferred_element_type=jnp.float32))

    return pl.pallas_call(
        body,
        grid=(pl.cdiv(_N, R), pl.cdiv(_N, C)),
        in_specs=[
            pl.BlockSpec((R, 32), lambda i, j: (i, 0)),
            pl.BlockSpec((32, C), lambda i, j: (0, j)),
        ],
        out_specs=pl.BlockSpec((R, C), lambda i, j: (i, j)),
        out_shape=jax.ShapeDtypeStruct((_N, _N), jnp.float32),
    )(haB, haT)


def _decX(z, W1, b1, W2, b2, W3, b3, Wpi, bpi, Wdp, bdp, Wmu, bmu):
    R = 1000

    def body(z_ref, W1_ref, b1_ref, W2_ref, b2_ref, W3_ref, b3_ref,
             Wpi_ref, bpi_ref, Wdp_ref, bdp_ref, Wmu_ref, bmu_ref,
             pi_ref, dp_ref, mu_ref):
        h = jnp.maximum(z_ref[...] @ W1_ref[...] + b1_ref[...], 0.0)
        h = jnp.maximum(h @ W2_ref[...] + b2_ref[...], 0.0)
        h = jnp.maximum(h @ W3_ref[...] + b3_ref[...], 0.0)
        pi_ref[...] = jax.nn.sigmoid(h @ Wpi_ref[...] + bpi_ref[...])
        sp = jax.nn.softplus(h @ Wdp_ref[...] + bdp_ref[...])
        dp_ref[...] = jnp.clip(sp, 1e-4, 1e4)
        mu_ref[...] = jnp.clip(jnp.exp(h @ Wmu_ref[...] + bmu_ref[...]),
                               1e-5, 1e6)

    full = lambda a, b: pl.BlockSpec((a, b), lambda i: (0, 0))
    ospec = pl.BlockSpec((R, 128), lambda i: (i, 0))
    oshape = jax.ShapeDtypeStruct((_N, 128), jnp.float32)
    return pl.pallas_call(
        body,
        grid=(_N // R,),
        in_specs=[
            pl.BlockSpec((R, 15), lambda i: (i, 0)),
            full(15, 128), full(1, 128),
            full(128, 256), full(1, 256),
            full(256, 512), full(1, 512),
            full(512, 128), full(1, 128),
            full(512, 128), full(1, 128),
            full(512, 128), full(1, 128),
        ],
        out_specs=[ospec, ospec, ospec],
        out_shape=[oshape, oshape, oshape],
    )(z, W1, b1, W2, b2, W3, b3, Wpi, bpi, Wdp, bdp, Wmu, bmu)


def kernel(x, edge_index, edge_weight, W_tag1, b_tag1, W_tag2, b_tag2,
           W_deca, b_deca, W_bil, W_dx1, b_dx1, W_dx2, b_dx2, W_dx3, b_dx3,
           W_pi, b_pi, W_disp, b_disp, W_mean, b_mean):
    src = edge_index[0].astype(jnp.int32)
    dst = edge_index[1].astype(jnp.int32)
    w = edge_weight
    z128 = jnp.zeros((_N, 128), jnp.float32)
    z16 = jnp.zeros((_N, 16), jnp.float32)

    p1 = _prop128(x, src, dst, w, z128)
    hop1 = _merge2(p1)
    p2 = _prop128(hop1, src, dst, w, z128)
    hop2 = _merge2(p2)
    p3 = _prop128(hop2, src, dst, w, z128)

    W2c = jnp.pad(
        W_tag2.reshape(4, 128, 15).transpose(1, 0, 2),
        ((0, 0), (0, 0), (0, 1))).reshape(128, 64)
    zw0, zw1, zw2, zw3 = _tag1_final(
        x, hop1, hop2, p3, W_tag1, b_tag1.reshape(1, 128), W2c)

    q3 = _prop16(zw3, src, dst, w, z16)
    m3 = _add3_16(zw2, q3)
    q2 = _prop16(m3, src, dst, w, z16)
    m2 = _add3_16(zw1, q2)
    q1 = _prop16(m2, src, dst, w, z16)
    z, ha, haB = _final_enc(zw0, q1, b_tag2.reshape(1, 15), W_deca,
                            b_deca.reshape(1, 32), W_bil)

    A_out = _decA(haB, ha.T)
    pi, disp, mean = _decX(
        z, W_dx1, b_dx1.reshape(1, 128), W_dx2, b_dx2.reshape(1, 256),
        W_dx3, b_dx3.reshape(1, 512), W_pi, b_pi.reshape(1, 128),
        W_disp, b_disp.reshape(1, 128), W_mean, b_mean.reshape(1, 128))
    return (z, A_out, pi, disp, mean)


# speedup vs baseline: 7.3517x; 1.0303x over previous
import functools

import jax
import jax.numpy as jnp
from jax import lax
from jax.experimental import pallas as pl
from jax.experimental.pallas import tpu as pltpu
from jax.experimental.pallas import tpu_sc as plsc

_N = 10000
_E = 160000
_NW = 32
_RPT = 624


def _make_prop(D, chunk):
    nch_total = _E // chunk
    nch_base = nch_total // _NW
    nch_extra = nch_total % _NW
    mesh = plsc.VectorSubcoreMesh(core_axis_name="c", subcore_axis_name="s",
                                  num_cores=2, num_subcores=16)

    @functools.partial(
        pl.kernel,
        out_type=jax.ShapeDtypeStruct((2, _N, D), jnp.float32),
        mesh=mesh,
        compiler_params=pltpu.CompilerParams(use_tc_tiling_on_sc=(D == 128)),
        scratch_types=[
            pltpu.VMEM((chunk,), jnp.int32),
            pltpu.VMEM((chunk,), jnp.int32),
            pltpu.VMEM((chunk,), jnp.int32),
            pltpu.VMEM((chunk,), jnp.int32),
            pltpu.VMEM((chunk,), jnp.float32),
            pltpu.VMEM((chunk,), jnp.float32),
            pltpu.VMEM((chunk, D), jnp.float32),
            pltpu.VMEM((chunk, D), jnp.float32),
            pltpu.VMEM_SHARED((_N, D), jnp.float32),
            pltpu.SemaphoreType.DMA,
            pltpu.SemaphoreType.DMA,
            pltpu.SemaphoreType.DMA,
            pltpu.SemaphoreType.DMA,
            pltpu.SemaphoreType.DMA,
            pltpu.SemaphoreType.DMA,
        ],
    )
    def prop(table, src, dst, w, zeros, out, src0, src1, dst0, dst1, w0, w1,
             rows0, rows1, acc_sh, sg0, sg1, ss0, ss1, si0, si1):
        src_s = (src0, src1)
        dst_s = (dst0, dst1)
        w_s = (w0, w1)
        rows_b = (rows0, rows1)
        sem_g = (sg0, sg1)
        sem_s = (ss0, ss1)
        sem_i = (si0, si1)
        cid = lax.axis_index("c")
        sid = lax.axis_index("s")
        wid = cid * 16 + sid
        ntrips = nch_base + jnp.where(wid < nch_extra, 1, 0)

        def load_idx(i, b):
            base = (wid + _NW * i) * chunk
            pltpu.sync_copy(src.at[pl.ds(base, chunk)], src_s[b])
            pltpu.sync_copy(dst.at[pl.ds(base, chunk)], dst_s[b])
            pltpu.sync_copy(w.at[pl.ds(base, chunk)], w_s[b])

        def idx_descs(i, b):
            base = (wid + _NW * i) * chunk
            return (
                pltpu.make_async_copy(src.at[pl.ds(base, chunk)], src_s[b],
                                      sem_i[b]),
                pltpu.make_async_copy(dst.at[pl.ds(base, chunk)], dst_s[b],
                                      sem_i[b]),
                pltpu.make_async_copy(w.at[pl.ds(base, chunk)], w_s[b],
                                      sem_i[b]),
            )

        def gather_desc(b):
            return pltpu.make_async_copy(table.at[src_s[b]], rows_b[b],
                                         sem_g[b])

        def scatter_desc(b):
            return pltpu.make_async_copy(rows_b[b], acc_sh.at[dst_s[b]],
                                         sem_s[b])

        def issue_scatter(b):
            pltpu.async_copy(rows_b[b], acc_sh.at[dst_s[b]], sem_s[b],
                             add=True)

        def scale(b):
            rv = rows_b[b]
            wref = w_s[b]

            def scale16(j, carry):
                w16 = wref[pl.ds(pl.multiple_of(j * 16, 8), 16)]
                for e in range(16):
                    ws = jnp.full((16,), w16[e], dtype=jnp.float32)
                    row = j * 16 + e
                    for c in range(D // 16):
                        rv[row, pl.ds(c * 16, 16)] = (
                            rv[row, pl.ds(c * 16, 16)] * ws)
                return carry

            lax.fori_loop(0, chunk // 16, scale16, 0)

        r0 = sid * _RPT
        pltpu.sync_copy(zeros.at[pl.ds(r0, _RPT)], acc_sh.at[pl.ds(r0, _RPT)])

        @pl.when(sid == 15)
        def _():
            tail = 16 * _RPT
            pltpu.sync_copy(zeros.at[pl.ds(tail, _N - tail)],
                            acc_sh.at[pl.ds(tail, _N - tail)])

        plsc.subcore_barrier()

        load_idx(0, 0)
        gather_desc(0).start()

        def step(i, carry):
            for b in (0, 1):
                @pl.when(i % 2 == b)
                def _(b=b):
                    nb = 1 - b

                    @pl.when(jnp.logical_and(i >= 1, i + 1 < ntrips))
                    def _():
                        scatter_desc(nb).wait()

                    @pl.when(i + 1 < ntrips)
                    def _():
                        for dsc in idx_descs(i + 1, nb):
                            dsc.start()

                    gather_desc(b).wait()
                    scale(b)
                    issue_scatter(b)

                    @pl.when(i + 1 < ntrips)
                    def _():
                        for dsc in idx_descs(i + 1, nb):
                            dsc.wait()
                        gather_desc(nb).start()
            return carry

        lax.fori_loop(0, ntrips, step, 0)
        scatter_desc(0).wait()
        scatter_desc(1).wait()

        plsc.subcore_barrier()
        pltpu.sync_copy(acc_sh.at[pl.ds(r0, _RPT)],
                        out.at[cid, pl.ds(r0, _RPT)])

        @pl.when(sid == 15)
        def _():
            tail = 16 * _RPT
            pltpu.sync_copy(acc_sh.at[pl.ds(tail, _N - tail)],
                            out.at[cid, pl.ds(tail, _N - tail)])

    return prop


@functools.lru_cache(maxsize=None)
def _get_prop(D, chunk):
    return _make_prop(D, chunk)


def _prop128(*args):
    return _get_prop(128, 160)(*args)


def _prop16(*args):
    return _get_prop(16, 1280)(*args)


def _merge2(p):
    R = 1000

    def body(p_ref, o_ref):
        o_ref[...] = p_ref[0] + p_ref[1]

    return pl.pallas_call(
        body,
        grid=(_N // R,),
        in_specs=[pl.BlockSpec((2, R, 128), lambda i: (0, i, 0))],
        out_specs=pl.BlockSpec((R, 128), lambda i: (i, 0)),
        out_shape=jax.ShapeDtypeStruct((_N, 128), jnp.float32),
    )(p)


def _tag1_final(x, h1, h2, p3, W1, b1, W2c):
    R = 1000

    def body(x_ref, h1_ref, h2_ref, p3_ref, W1_ref, b1_ref, W2c_ref,
             z0_ref, z1_ref, z2_ref, z3_ref):
        hop3 = p3_ref[0] + p3_ref[1]
        acc = x_ref[...] @ W1_ref[0:128]
        acc += h1_ref[...] @ W1_ref[128:256]
        acc += h2_ref[...] @ W1_ref[256:384]
        acc += hop3 @ W1_ref[384:512]
        h = jnp.maximum(acc + b1_ref[...], 0.0)
        zw = h @ W2c_ref[...]
        z0_ref[...] = zw[:, 0:16]
        z1_ref[...] = zw[:, 16:32]
        z2_ref[...] = zw[:, 32:48]
        z3_ref[...] = zw[:, 48:64]

    zspec = pl.BlockSpec((R, 16), lambda i: (i, 0))
    zshape = jax.ShapeDtypeStruct((_N, 16), jnp.float32)
    return pl.pallas_call(
        body,
        grid=(_N // R,),
        in_specs=[
            pl.BlockSpec((R, 128), lambda i: (i, 0)),
            pl.BlockSpec((R, 128), lambda i: (i, 0)),
            pl.BlockSpec((R, 128), lambda i: (i, 0)),
            pl.BlockSpec((2, R, 128), lambda i: (0, i, 0)),
            pl.BlockSpec((512, 128), lambda i: (0, 0)),
            pl.BlockSpec((1, 128), lambda i: (0, 0)),
            pl.BlockSpec((128, 64), lambda i: (0, 0)),
        ],
        out_specs=[zspec, zspec, zspec, zspec],
        out_shape=[zshape, zshape, zshape, zshape],
    )(x, h1, h2, p3, W1, b1, W2c)


def _add3_16(a, p):
    R = 2000

    def body(a_ref, p_ref, o_ref):
        o_ref[...] = a_ref[...] + p_ref[0] + p_ref[1]

    return pl.pallas_call(
        body,
        grid=(_N // R,),
        in_specs=[
            pl.BlockSpec((R, 16), lambda i: (i, 0)),
            pl.BlockSpec((2, R, 16), lambda i: (0, i, 0)),
        ],
        out_specs=pl.BlockSpec((R, 16), lambda i: (i, 0)),
        out_shape=jax.ShapeDtypeStruct((_N, 16), jnp.float32),
    )(a, p)


def _final_enc(zw0, q1, b2, Wd, bd, Wb):
    R = 2000

    def body(zw0_ref, q1_ref, b2_ref, Wd_ref, bd_ref, Wb_ref,
             z_ref, ha_ref, haB_ref):
        zf = zw0_ref[...] + q1_ref[0] + q1_ref[1]
        z15 = zf[:, 0:15] + b2_ref[...]
        z_ref[...] = z15
        ha = z15 @ Wd_ref[...] + bd_ref[...]
        ha_ref[...] = ha.astype(jnp.bfloat16)
        haB_ref[...] = (ha @ Wb_ref[...]).astype(jnp.bfloat16)

    return pl.pallas_call(
        body,
        grid=(_N // R,),
        in_specs=[
            pl.BlockSpec((R, 16), lambda i: (i, 0)),
            pl.BlockSpec((2, R, 16), lambda i: (0, i, 0)),
            pl.BlockSpec((1, 15), lambda i: (0, 0)),
            pl.BlockSpec((15, 32), lambda i: (0, 0)),
            pl.BlockSpec((1, 32), lambda i: (0, 0)),
            pl.BlockSpec((32, 32), lambda i: (0, 0)),
        ],
        out_specs=[
            pl.BlockSpec((R, 15), lambda i: (i, 0)),
            pl.BlockSpec((R, 32), lambda i: (i, 0)),
            pl.BlockSpec((R, 32), lambda i: (i, 0)),
        ],
        out_shape=[
            jax.ShapeDtypeStruct((_N, 15), jnp.float32),
            jax.ShapeDtypeStruct((_N, 32), jnp.bfloat16),
            jax.ShapeDtypeStruct((_N, 32), jnp.bfloat16),
        ],
    )(zw0, q1, b2, Wd, bd, Wb)


def _decA(haB, haT):
    R, C = 1024, 1024

    def body(haB_ref, haT_ref, o_ref):
        logits = jnp.dot(haB_ref[...], haT_ref[...],
                         preferred_element_type=jnp.float32)
        o_ref[...] = 0.5 * jnp.tanh(0.5 * logits) + 0.5

    return pl.pallas_call(
        body,
        grid=(pl.cdiv(_N, R), pl.cdiv(_N, C)),
        in_specs=[
            pl.BlockSpec((R, 32), lambda i, j: (i, 0)),
            pl.BlockSpec((32, C), lambda i, j: (0, j)),
        ],
        out_specs=pl.BlockSpec((R, C), lambda i, j: (i, j)),
        out_shape=jax.ShapeDtypeStruct((_N, _N), jnp.float32),
    )(haB, haT)


def _decX(z, W1, b1, W2, b2, W3, b3, Wpi, bpi, Wdp, bdp, Wmu, bmu):
    R = 1000

    def body(z_ref, W1_ref, b1_ref, W2_ref, b2_ref, W3_ref, b3_ref,
             Wpi_ref, bpi_ref, Wdp_ref, bdp_ref, Wmu_ref, bmu_ref,
             pi_ref, dp_ref, mu_ref):
        h = jnp.maximum(z_ref[...] @ W1_ref[...] + b1_ref[...], 0.0)
        h = jnp.maximum(h @ W2_ref[...] + b2_ref[...], 0.0)
        h = jnp.maximum(h @ W3_ref[...] + b3_ref[...], 0.0)
        pi_ref[...] = jax.nn.sigmoid(h @ Wpi_ref[...] + bpi_ref[...])
        sp = jax.nn.softplus(h @ Wdp_ref[...] + bdp_ref[...])
        dp_ref[...] = jnp.clip(sp, 1e-4, 1e4)
        mu_ref[...] = jnp.clip(jnp.exp(h @ Wmu_ref[...] + bmu_ref[...]),
                               1e-5, 1e6)

    full = lambda a, b: pl.BlockSpec((a, b), lambda i: (0, 0))
    ospec = pl.BlockSpec((R, 128), lambda i: (i, 0))
    oshape = jax.ShapeDtypeStruct((_N, 128), jnp.float32)
    return pl.pallas_call(
        body,
        grid=(_N // R,),
        in_specs=[
            pl.BlockSpec((R, 15), lambda i: (i, 0)),
            full(15, 128), full(1, 128),
            full(128, 256), full(1, 256),
            full(256, 512), full(1, 512),
            full(512, 128), full(1, 128),
            full(512, 128), full(1, 128),
            full(512, 128), full(1, 128),
        ],
        out_specs=[ospec, ospec, ospec],
        out_shape=[oshape, oshape, oshape],
    )(z, W1, b1, W2, b2, W3, b3, Wpi, bpi, Wdp, bdp, Wmu, bmu)


def kernel(x, edge_index, edge_weight, W_tag1, b_tag1, W_tag2, b_tag2,
           W_deca, b_deca, W_bil, W_dx1, b_dx1, W_dx2, b_dx2, W_dx3, b_dx3,
           W_pi, b_pi, W_disp, b_disp, W_mean, b_mean):
    src = edge_index[0].astype(jnp.int32)
    dst = edge_index[1].astype(jnp.int32)
    w = edge_weight
    z128 = jnp.zeros((_N, 128), jnp.float32)
    z16 = jnp.zeros((_N, 16), jnp.float32)

    p1 = _prop128(x, src, dst, w, z128)
    hop1 = _merge2(p1)
    p2 = _prop128(hop1, src, dst, w, z128)
    hop2 = _merge2(p2)
    p3 = _prop128(hop2, src, dst, w, z128)

    W2c = jnp.pad(
        W_tag2.reshape(4, 128, 15).transpose(1, 0, 2),
        ((0, 0), (0, 0), (0, 1))).reshape(128, 64)
    zw0, zw1, zw2, zw3 = _tag1_final(
        x, hop1, hop2, p3, W_tag1, b_tag1.reshape(1, 128), W2c)

    q3 = _prop16(zw3, src, dst, w, z16)
    m3 = _add3_16(zw2, q3)
    q2 = _prop16(m3, src, dst, w, z16)
    m2 = _add3_16(zw1, q2)
    q1 = _prop16(m2, src, dst, w, z16)
    z, ha, haB = _final_enc(zw0, q1, b_tag2.reshape(1, 15), W_deca,
                            b_deca.reshape(1, 32), W_bil)

    A_out = _decA(haB, ha.T)
    pi, disp, mean = _decX(
        z, W_dx1, b_dx1.reshape(1, 128), W_dx2, b_dx2.reshape(1, 256),
        W_dx3, b_dx3.reshape(1, 512), W_pi, b_pi.reshape(1, 128),
        W_disp, b_disp.reshape(1, 128), W_mean, b_mean.reshape(1, 128))
    return (z, A_out, pi, disp, mean)


# speedup vs baseline: 8.4175x; 1.1450x over previous
import functools

import jax
import jax.numpy as jnp
from jax import lax
from jax.experimental import pallas as pl
from jax.experimental.pallas import tpu as pltpu
from jax.experimental.pallas import tpu_sc as plsc

_N = 10000
_E = 160000
_NW = 32
_RPT = 624


def _make_prop(D, chunk):
    nch_total = _E // chunk
    nch_base = nch_total // _NW
    nch_extra = nch_total % _NW
    mesh = plsc.VectorSubcoreMesh(core_axis_name="c", subcore_axis_name="s",
                                  num_cores=2, num_subcores=16)

    @functools.partial(
        pl.kernel,
        out_type=jax.ShapeDtypeStruct((2, _N, D), jnp.float32),
        mesh=mesh,
        compiler_params=pltpu.CompilerParams(use_tc_tiling_on_sc=(D == 128)),
        scratch_types=[
            pltpu.VMEM((chunk,), jnp.int32),
            pltpu.VMEM((chunk,), jnp.int32),
            pltpu.VMEM((chunk,), jnp.int32),
            pltpu.VMEM((chunk,), jnp.int32),
            pltpu.VMEM((chunk,), jnp.int32),
            pltpu.VMEM((chunk,), jnp.int32),
            pltpu.VMEM((chunk,), jnp.float32),
            pltpu.VMEM((chunk,), jnp.float32),
            pltpu.VMEM((chunk,), jnp.float32),
            pltpu.VMEM((chunk,), jnp.float32),
            pltpu.VMEM((chunk, D), jnp.float32),
            pltpu.VMEM((chunk, D), jnp.float32),
            pltpu.VMEM_SHARED((_N, D), jnp.float32),
            pltpu.SemaphoreType.DMA,
            pltpu.SemaphoreType.DMA,
            pltpu.SemaphoreType.DMA,
            pltpu.SemaphoreType.DMA,
            pltpu.SemaphoreType.DMA,
            pltpu.SemaphoreType.DMA,
            pltpu.SemaphoreType.DMA,
        ],
    )
    def prop(table, src, dst, w, zeros, out, src0, src1, src2,
             dst0, dst1, dst2, w0, w1, w2, w_cur, rows0, rows1,
             acc_sh, sg0, sg1, ss0, ss1, si0, si1, si2):
        src_s = (src0, src1, src2)
        dst_s = (dst0, dst1, dst2)
        w_s = (w0, w1, w2)
        rows_b = (rows0, rows1)
        sem_g = (sg0, sg1)
        sem_s = (ss0, ss1)
        sem_i = (si0, si1, si2)
        cid = lax.axis_index("c")
        sid = lax.axis_index("s")
        wid = cid * 16 + sid
        ntrips = nch_base + jnp.where(wid < nch_extra, 1, 0)

        def idx_descs(i, t):
            base = (wid + _NW * i) * chunk
            return (
                pltpu.make_async_copy(src.at[pl.ds(base, chunk)], src_s[t],
                                      sem_i[t]),
                pltpu.make_async_copy(dst.at[pl.ds(base, chunk)], dst_s[t],
                                      sem_i[t]),
                pltpu.make_async_copy(w.at[pl.ds(base, chunk)], w_s[t],
                                      sem_i[t]),
            )

        def gather_desc(b, t):
            return pltpu.make_async_copy(table.at[src_s[t]], rows_b[b],
                                         sem_g[b])

        def scatter_desc(b, t):
            return pltpu.make_async_copy(rows_b[b], acc_sh.at[dst_s[t]],
                                         sem_s[b])

        def scale(b):
            rv = rows_b[b]

            def scale16(j, carry):
                w16 = w_cur[pl.ds(pl.multiple_of(j * 16, 8), 16)]
                for e in range(16):
                    ws = jnp.full((16,), w16[e], dtype=jnp.float32)
                    row = j * 16 + e
                    for c in range(D // 16):
                        rv[row, pl.ds(c * 16, 16)] = (
                            rv[row, pl.ds(c * 16, 16)] * ws)
                return carry

            lax.fori_loop(0, chunk // 16, scale16, 0)

        for dsc in idx_descs(0, 0):
            dsc.start()
        for dsc in idx_descs(0, 0):
            dsc.wait()
        gather_desc(0, 0).start()

        @pl.when(ntrips >= 2)
        def _():
            for dsc in idx_descs(1, 1):
                dsc.start()

        r0 = sid * _RPT
        pltpu.sync_copy(zeros.at[pl.ds(r0, _RPT)], acc_sh.at[pl.ds(r0, _RPT)])

        @pl.when(sid == 15)
        def _():
            tail = 16 * _RPT
            pltpu.sync_copy(zeros.at[pl.ds(tail, _N - tail)],
                            acc_sh.at[pl.ds(tail, _N - tail)])

        plsc.subcore_barrier()

        def step(i, carry):
            for b in (0, 1):
                @pl.when(i % 2 == b)
                def _(b=b):
                    nb = 1 - b

                    @pl.when(jnp.logical_and(i >= 1, i + 1 < ntrips))
                    def _():
                        scatter_desc(nb, 0).wait()

                    for t in (0, 1, 2):
                        @pl.when(jnp.logical_and(i + 1 < ntrips,
                                                 (i + 1) % 3 == t))
                        def _(t=t):
                            for dsc in idx_descs(i + 1, t):
                                dsc.wait()
                            gather_desc(nb, t).start()

                    for t in (0, 1, 2):
                        @pl.when(i % 3 == t)
                        def _(t=t):
                            def wcopy(j, carry2):
                                s = pl.ds(pl.multiple_of(j * 16, 8), 16)
                                w_cur[s] = w_s[t][s]
                                return carry2
                            lax.fori_loop(0, chunk // 16, wcopy, 0)

                    gather_desc(b, 0).wait()
                    scale(b)
                    for t in (0, 1, 2):
                        @pl.when(i % 3 == t)
                        def _(t=t):
                            pltpu.async_copy(rows_b[b],
                                             acc_sh.at[dst_s[t]],
                                             sem_s[b], add=True)

                    @pl.when(i + 2 < ntrips)
                    def _():
                        for t in (0, 1, 2):
                            @pl.when((i + 2) % 3 == t)
                            def _(t=t):
                                for dsc in idx_descs(i + 2, t):
                                    dsc.start()
            return carry

        lax.fori_loop(0, ntrips, step, 0)
        scatter_desc(0, 0).wait()
        scatter_desc(1, 0).wait()

        plsc.subcore_barrier()
        pltpu.sync_copy(acc_sh.at[pl.ds(r0, _RPT)],
                        out.at[cid, pl.ds(r0, _RPT)])

        @pl.when(sid == 15)
        def _():
            tail = 16 * _RPT
            pltpu.sync_copy(acc_sh.at[pl.ds(tail, _N - tail)],
                            out.at[cid, pl.ds(tail, _N - tail)])

    return prop


@functools.lru_cache(maxsize=None)
def _get_prop(D, chunk):
    return _make_prop(D, chunk)


def _prop128(*args):
    return _get_prop(128, 160)(*args)


def _prop16(*args):
    return _get_prop(16, 1280)(*args)


def _merge2(p):
    R = 1000

    def body(p_ref, o_ref):
        o_ref[...] = p_ref[0] + p_ref[1]

    return pl.pallas_call(
        body,
        grid=(_N // R,),
        in_specs=[pl.BlockSpec((2, R, 128), lambda i: (0, i, 0))],
        out_specs=pl.BlockSpec((R, 128), lambda i: (i, 0)),
        out_shape=jax.ShapeDtypeStruct((_N, 128), jnp.float32),
    )(p)


def _tag1_final(x, h1, h2, p3, W1, b1, W2c):
    R = 1000

    def body(x_ref, h1_ref, h2_ref, p3_ref, W1_ref, b1_ref, W2c_ref,
             z0_ref, z1_ref, z2_ref, z3_ref):
        hop3 = p3_ref[0] + p3_ref[1]
        acc = x_ref[...] @ W1_ref[0:128]
        acc += h1_ref[...] @ W1_ref[128:256]
        acc += h2_ref[...] @ W1_ref[256:384]
        acc += hop3 @ W1_ref[384:512]
        h = jnp.maximum(acc + b1_ref[...], 0.0)
        zw = h @ W2c_ref[...]
        z0_ref[...] = zw[:, 0:16]
        z1_ref[...] = zw[:, 16:32]
        z2_ref[...] = zw[:, 32:48]
        z3_ref[...] = zw[:, 48:64]

    zspec = pl.BlockSpec((R, 16), lambda i: (i, 0))
    zshape = jax.ShapeDtypeStruct((_N, 16), jnp.float32)
    return pl.pallas_call(
        body,
        grid=(_N // R,),
        in_specs=[
            pl.BlockSpec((R, 128), lambda i: (i, 0)),
            pl.BlockSpec((R, 128), lambda i: (i, 0)),
            pl.BlockSpec((R, 128), lambda i: (i, 0)),
            pl.BlockSpec((2, R, 128), lambda i: (0, i, 0)),
            pl.BlockSpec((512, 128), lambda i: (0, 0)),
            pl.BlockSpec((1, 128), lambda i: (0, 0)),
            pl.BlockSpec((128, 64), lambda i: (0, 0)),
        ],
        out_specs=[zspec, zspec, zspec, zspec],
        out_shape=[zshape, zshape, zshape, zshape],
    )(x, h1, h2, p3, W1, b1, W2c)


def _add3_16(a, p):
    R = 2000

    def body(a_ref, p_ref, o_ref):
        o_ref[...] = a_ref[...] + p_ref[0] + p_ref[1]

    return pl.pallas_call(
        body,
        grid=(_N // R,),
        in_specs=[
            pl.BlockSpec((R, 16), lambda i: (i, 0)),
            pl.BlockSpec((2, R, 16), lambda i: (0, i, 0)),
        ],
        out_specs=pl.BlockSpec((R, 16), lambda i: (i, 0)),
        out_shape=jax.ShapeDtypeStruct((_N, 16), jnp.float32),
    )(a, p)


def _final_enc(zw0, q1, b2, Wd, bd, Wb):
    R = 2000

    def body(zw0_ref, q1_ref, b2_ref, Wd_ref, bd_ref, Wb_ref,
             z_ref, ha_ref, haB_ref):
        zf = zw0_ref[...] + q1_ref[0] + q1_ref[1]
        z15 = zf[:, 0:15] + b2_ref[...]
        z_ref[...] = z15
        ha = z15 @ Wd_ref[...] + bd_ref[...]
        ha_ref[...] = ha.astype(jnp.bfloat16)
        haB_ref[...] = (ha @ Wb_ref[...]).astype(jnp.bfloat16)

    return pl.pallas_call(
        body,
        grid=(_N // R,),
        in_specs=[
            pl.BlockSpec((R, 16), lambda i: (i, 0)),
            pl.BlockSpec((2, R, 16), lambda i: (0, i, 0)),
            pl.BlockSpec((1, 15), lambda i: (0, 0)),
            pl.BlockSpec((15, 32), lambda i: (0, 0)),
            pl.BlockSpec((1, 32), lambda i: (0, 0)),
            pl.BlockSpec((32, 32), lambda i: (0, 0)),
        ],
        out_specs=[
            pl.BlockSpec((R, 15), lambda i: (i, 0)),
            pl.BlockSpec((R, 32), lambda i: (i, 0)),
            pl.BlockSpec((R, 32), lambda i: (i, 0)),
        ],
        out_shape=[
            jax.ShapeDtypeStruct((_N, 15), jnp.float32),
            jax.ShapeDtypeStruct((_N, 32), jnp.bfloat16),
            jax.ShapeDtypeStruct((_N, 32), jnp.bfloat16),
        ],
    )(zw0, q1, b2, Wd, bd, Wb)


def _decA(haB, haT):
    R, C = 1024, 1024

    def body(haB_ref, haT_ref, o_ref):
        logits = jnp.dot(haB_ref[...], haT_ref[...],
                         preferred_element_type=jnp.float32)
        o_ref[...] = 0.5 * jnp.tanh(0.5 * logits) + 0.5

    return pl.pallas_call(
        body,
        grid=(pl.cdiv(_N, R), pl.cdiv(_N, C)),
        in_specs=[
            pl.BlockSpec((R, 32), lambda i, j: (i, 0)),
            pl.BlockSpec((32, C), lambda i, j: (0, j)),
        ],
        out_specs=pl.BlockSpec((R, C), lambda i, j: (i, j)),
        out_shape=jax.ShapeDtypeStruct((_N, _N), jnp.float32),
    )(haB, haT)


def _decX(z, W1, b1, W2, b2, W3, b3, Wpi, bpi, Wdp, bdp, Wmu, bmu):
    R = 1000

    def body(z_ref, W1_ref, b1_ref, W2_ref, b2_ref, W3_ref, b3_ref,
             Wpi_ref, bpi_ref, Wdp_ref, bdp_ref, Wmu_ref, bmu_ref,
             pi_ref, dp_ref, mu_ref):
        h = jnp.maximum(z_ref[...] @ W1_ref[...] + b1_ref[...], 0.0)
        h = jnp.maximum(h @ W2_ref[...] + b2_ref[...], 0.0)
        h = jnp.maximum(h @ W3_ref[...] + b3_ref[...], 0.0)
        pi_ref[...] = jax.nn.sigmoid(h @ Wpi_ref[...] + bpi_ref[...])
        sp = jax.nn.softplus(h @ Wdp_ref[...] + bdp_ref[...])
        dp_ref[...] = jnp.clip(sp, 1e-4, 1e4)
        mu_ref[...] = jnp.clip(jnp.exp(h @ Wmu_ref[...] + bmu_ref[...]),
                               1e-5, 1e6)

    full = lambda a, b: pl.BlockSpec((a, b), lambda i: (0, 0))
    ospec = pl.BlockSpec((R, 128), lambda i: (i, 0))
    oshape = jax.ShapeDtypeStruct((_N, 128), jnp.float32)
    return pl.pallas_call(
        body,
        grid=(_N // R,),
        in_specs=[
            pl.BlockSpec((R, 15), lambda i: (i, 0)),
            full(15, 128), full(1, 128),
            full(128, 256), full(1, 256),
            full(256, 512), full(1, 512),
            full(512, 128), full(1, 128),
            full(512, 128), full(1, 128),
            full(512, 128), full(1, 128),
        ],
        out_specs=[ospec, ospec, ospec],
        out_shape=[oshape, oshape, oshape],
    )(z, W1, b1, W2, b2, W3, b3, Wpi, bpi, Wdp, bdp, Wmu, bmu)


def kernel(x, edge_index, edge_weight, W_tag1, b_tag1, W_tag2, b_tag2,
           W_deca, b_deca, W_bil, W_dx1, b_dx1, W_dx2, b_dx2, W_dx3, b_dx3,
           W_pi, b_pi, W_disp, b_disp, W_mean, b_mean):
    src = edge_index[0].astype(jnp.int32)
    dst = edge_index[1].astype(jnp.int32)
    w = edge_weight
    z128 = jnp.zeros((_N, 128), jnp.float32)
    z16 = jnp.zeros((_N, 16), jnp.float32)

    p1 = _prop128(x, src, dst, w, z128)
    hop1 = _merge2(p1)
    p2 = _prop128(hop1, src, dst, w, z128)
    hop2 = _merge2(p2)
    p3 = _prop128(hop2, src, dst, w, z128)

    W2c = jnp.pad(
        W_tag2.reshape(4, 128, 15).transpose(1, 0, 2),
        ((0, 0), (0, 0), (0, 1))).reshape(128, 64)
    zw0, zw1, zw2, zw3 = _tag1_final(
        x, hop1, hop2, p3, W_tag1, b_tag1.reshape(1, 128), W2c)

    q3 = _prop16(zw3, src, dst, w, z16)
    m3 = _add3_16(zw2, q3)
    q2 = _prop16(m3, src, dst, w, z16)
    m2 = _add3_16(zw1, q2)
    q1 = _prop16(m2, src, dst, w, z16)
    z, ha, haB = _final_enc(zw0, q1, b_tag2.reshape(1, 15), W_deca,
                            b_deca.reshape(1, 32), W_bil)

    A_out = _decA(haB, ha.T)
    pi, disp, mean = _decX(
        z, W_dx1, b_dx1.reshape(1, 128), W_dx2, b_dx2.reshape(1, 256),
        W_dx3, b_dx3.reshape(1, 512), W_pi, b_pi.reshape(1, 128),
        W_disp, b_disp.reshape(1, 128), W_mean, b_mean.reshape(1, 128))
    return (z, A_out, pi, disp, mean)
